# baseline (speedup 1.0000x reference)
import functools

import jax
import jax.numpy as jnp
from jax import lax
from jax.experimental import pallas as pl
from jax.experimental.pallas import tpu as pltpu
from jax.experimental.pallas import tpu_sc as plsc

_H = 20
_WD = 20
_F = 32
_K = 8
_N = _H * _WD
_NC = 2
_NS = 16
_CELLS_PER_W = _WD
_ROWS_PER_W = _CELLS_PER_W * _K
_GATHER_SPLIT = 80
_LANES = 16
_SLAB = 24


def _prep_body(x_ref, w_ref, nn_ref, xw_ref, meta_ref):
    w = w_ref[...]
    for i in range(_H):
        xw_ref[pl.ds(i * _WD, _WD), pl.ds(0, _F)] = jnp.dot(
            x_ref[i], w, preferred_element_type=jnp.float32)
    nnv = nn_ref[...]
    cc = lax.broadcasted_iota(jnp.int32, nnv.shape, 3)
    zero = jnp.float32(0.0)
    ids_f = jnp.sum(jnp.where(cc == 1, nnv, zero), axis=-1)
    sims = jnp.sum(jnp.where(cc == 2, nnv, zero), axis=-1)
    id_bits = lax.bitcast_convert_type(ids_f.astype(jnp.int32), jnp.float32)
    ii = lax.broadcasted_iota(jnp.int32, (_H, _WD, _K), 0)
    jj = lax.broadcasted_iota(jnp.int32, (_H, _WD, _K), 1)
    kk = lax.broadcasted_iota(jnp.int32, (_H, _WD, _K), 2)
    valid = (ii < _H - 1) & (jj < _WD - 1) & (kk < _K - 1)
    sims = jnp.where(valid, sims, jnp.float32(0.0))
    cat = jnp.concatenate([id_bits, sims], axis=-1)
    for i in range(_H):
        meta_ref[pl.ds(i * _SLAB, _WD), pl.ds(0, 2 * _K)] = cat[i]


def _prep(x, w, nn):
    return pl.pallas_call(
        _prep_body,
        out_shape=[
            jax.ShapeDtypeStruct((_N, 128), jnp.float32),
            jax.ShapeDtypeStruct((_H * _SLAB, 128), jnp.float32),
        ],
    )(x, w, nn)


def _sc_body(xw_hbm, meta_hbm, out_hbm, metav, idx_v, rows_v, out_v, sem):
    wid = lax.axis_index("s") * _NC + lax.axis_index("c")

    @pl.when(wid < _H)
    def _():
        pltpu.sync_copy(meta_hbm.at[pl.ds(wid * _SLAB, _SLAB)], metav)
        lane = lax.iota(jnp.int32, _LANES)
        lane_k = lane & (_K - 1)
        half_cell = lane >> 3
        sim_regs = []
        for v in range(_ROWS_PER_W // _LANES):
            row_vec = 2 * v + half_cell
            idb = plsc.load_gather(metav, [row_vec, lane_k])
            sim_regs.append(plsc.load_gather(metav, [row_vec, lane_k + _K]))
            idx_v[pl.ds(v * _LANES, _LANES)] = plsc.bitcast(idb, jnp.int32)
        cp0 = pltpu.async_copy(
            xw_hbm.at[idx_v.at[pl.ds(0, _GATHER_SPLIT)]],
            rows_v.at[pl.ds(0, _GATHER_SPLIT)], sem)
        cp1 = pltpu.async_copy(
            xw_hbm.at[idx_v.at[pl.ds(_GATHER_SPLIT, _GATHER_SPLIT)]],
            rows_v.at[pl.ds(_GATHER_SPLIT, _GATHER_SPLIT)], sem)
        cp0.wait()
        cp1.wait()
        for v in range(_ROWS_PER_W // _LANES):
            sv = sim_regs[v]
            for half, c in ((0, 2 * v), (_K, 2 * v + 1)):
                r0 = c * _K
                s = sv[half]
                acc_lo = s * rows_v[r0, pl.ds(0, _LANES)]
                acc_hi = s * rows_v[r0, pl.ds(_LANES, _LANES)]
                for k in range(1, _K):
                    r = r0 + k
                    s = sv[half + k]
                    acc_lo = acc_lo + s * rows_v[r, pl.ds(0, _LANES)]
                    acc_hi = acc_hi + s * rows_v[r, pl.ds(_LANES, _LANES)]
                out_v[c, pl.ds(0, _LANES)] = acc_lo
                out_v[c, pl.ds(_LANES, _LANES)] = acc_hi
        pltpu.sync_copy(out_v, out_hbm.at[wid])


_sc_gather = functools.partial(
    pl.kernel,
    out_type=jax.ShapeDtypeStruct((_H, _WD, _F), jnp.float32),
    mesh=plsc.VectorSubcoreMesh(core_axis_name="c", subcore_axis_name="s",
                                num_cores=_NC, num_subcores=_NS),
    scratch_types=[
        pltpu.VMEM((_SLAB, 128), jnp.float32),
        pltpu.VMEM((_ROWS_PER_W,), jnp.int32),
        pltpu.VMEM((_ROWS_PER_W, 128), jnp.float32),
        pltpu.VMEM((_CELLS_PER_W, _F), jnp.float32),
        pltpu.SemaphoreType.DMA,
    ],
    compiler_params=pltpu.CompilerParams(use_tc_tiling_on_sc=True,
                                         needs_layout_passes=False),
)(_sc_body)


def kernel(x, W, nearest_neighbors):
    xw, meta = _prep(x, W, nearest_neighbors)
    return _sc_gather(xw, meta)


# speedup vs baseline: 1.0212x; 1.0212x over previous
import functools

import jax
import jax.numpy as jnp
from jax import lax
from jax.experimental import pallas as pl
from jax.experimental.pallas import tpu as pltpu
from jax.experimental.pallas import tpu_sc as plsc

_H = 20
_WD = 20
_F = 32
_K = 8
_N = _H * _WD
_NC = 2
_NS = 16
_CELLS_PER_W = _WD
_ROWS_PER_W = _CELLS_PER_W * _K
_GATHER_SPLIT = 80
_LANES = 16
_SLAB = 24


def _mm_body(x_ref, w_ref, xw_ref):
    w = w_ref[...]
    for i in range(_H):
        xw_ref[pl.ds(i * _WD, _WD), pl.ds(0, _F)] = jnp.dot(
            x_ref[i], w, preferred_element_type=jnp.float32)


def _project(x, w):
    return pl.pallas_call(
        _mm_body,
        out_shape=jax.ShapeDtypeStruct((_N, 128), jnp.float32),
    )(x, w)


def _sc_body(xw_hbm, nn_hbm, out_hbm, nnv, idx_v, rows_v, out_v, sem):
    wid = lax.axis_index("s") * _NC + lax.axis_index("c")

    @pl.when(wid < _H)
    def _():
        pltpu.sync_copy(nn_hbm.at[wid], nnv)
        lane = lax.iota(jnp.int32, _LANES)
        lane_k = lane & (_K - 1)
        half_cell = lane >> 3
        kvalid = lane_k < _K - 1
        sim_regs = []
        for v in range(_ROWS_PER_W // _LANES):
            a_vec = 2 * v + half_cell
            idf = plsc.load_gather(
                nnv, [a_vec, lane_k, jnp.full((_LANES,), 1, jnp.int32)])
            sif = plsc.load_gather(
                nnv, [a_vec, lane_k, jnp.full((_LANES,), 2, jnp.int32)])
            valid = kvalid & (a_vec < _WD - 1) & (wid < _H - 1)
            sim_regs.append(jnp.where(valid, sif, jnp.float32(0.0)))
            idx_v[pl.ds(v * _LANES, _LANES)] = idf.astype(jnp.int32)
        cp0 = pltpu.async_copy(
            xw_hbm.at[idx_v.at[pl.ds(0, _GATHER_SPLIT)]],
            rows_v.at[pl.ds(0, _GATHER_SPLIT)], sem)
        cp1 = pltpu.async_copy(
            xw_hbm.at[idx_v.at[pl.ds(_GATHER_SPLIT, _GATHER_SPLIT)]],
            rows_v.at[pl.ds(_GATHER_SPLIT, _GATHER_SPLIT)], sem)
        cp0.wait()
        cp1.wait()
        for v in range(_ROWS_PER_W // _LANES):
            sv = sim_regs[v]
            for half, c in ((0, 2 * v), (_K, 2 * v + 1)):
                r0 = c * _K
                s = sv[half]
                acc_lo = s * rows_v[r0, pl.ds(0, _LANES)]
                acc_hi = s * rows_v[r0, pl.ds(_LANES, _LANES)]
                for k in range(1, _K):
                    r = r0 + k
                    s = sv[half + k]
                    acc_lo = acc_lo + s * rows_v[r, pl.ds(0, _LANES)]
                    acc_hi = acc_hi + s * rows_v[r, pl.ds(_LANES, _LANES)]
                out_v[c, pl.ds(0, _LANES)] = acc_lo
                out_v[c, pl.ds(_LANES, _LANES)] = acc_hi
        pltpu.sync_copy(out_v, out_hbm.at[wid])


_sc_gather = functools.partial(
    pl.kernel,
    out_type=jax.ShapeDtypeStruct((_H, _WD, _F), jnp.float32),
    mesh=plsc.VectorSubcoreMesh(core_axis_name="c", subcore_axis_name="s",
                                num_cores=_NC, num_subcores=_NS),
    scratch_types=[
        pltpu.VMEM((_CELLS_PER_W, _K, 3), jnp.float32),
        pltpu.VMEM((_ROWS_PER_W,), jnp.int32),
        pltpu.VMEM((_ROWS_PER_W, 128), jnp.float32),
        pltpu.VMEM((_CELLS_PER_W, _F), jnp.float32),
        pltpu.SemaphoreType.DMA,
    ],
    compiler_params=pltpu.CompilerParams(use_tc_tiling_on_sc=True,
                                         needs_layout_passes=False),
)(_sc_body)


def kernel(x, W, nearest_neighbors):
    xw = _project(x, W)
    return _sc_gather(xw, nearest_neighbors)


# speedup vs baseline: 1.1343x; 1.1108x over previous
import functools

import jax
import jax.numpy as jnp
from jax import lax
from jax.experimental import pallas as pl
from jax.experimental.pallas import tpu as pltpu
from jax.experimental.pallas import tpu_sc as plsc

_H = 20
_WD = 20
_F = 32
_K = 8
_N = _H * _WD
_NC = 2
_NS = 16
_CELLS_PER_W = _WD
_ROWS_PER_W = _CELLS_PER_W * _K
_GATHER_SPLIT = 80
_LANES = 16
_SLAB = 24


def _mm_body(x_ref, w_ref, xw_ref):
    w = w_ref[...]
    for i in range(_H):
        xw_ref[pl.ds(i * _WD, _WD), pl.ds(0, _F)] = jnp.dot(
            x_ref[i], w, preferred_element_type=jnp.float32)


def _project(x, w):
    return pl.pallas_call(
        _mm_body,
        out_shape=jax.ShapeDtypeStruct((_N, 128), jnp.float32),
    )(x, w)


def _sc_body(xw_hbm, nn_hbm, out_hbm, nnv, idx_v, rows_v, out_v, sem):
    wid = lax.axis_index("s") * _NC + lax.axis_index("c")

    @pl.when(wid < _H)
    def _():
        pltpu.sync_copy(nn_hbm.at[wid], nnv)
        lane = lax.iota(jnp.int32, _LANES)
        lane_k = lane & (_K - 1)
        half_cell = lane >> 3
        kvalid = lane_k < _K - 1
        one = jnp.full((_LANES,), 1, jnp.int32)
        sim_regs = []
        for v in range(_ROWS_PER_W // _LANES):
            a_vec = 2 * v + half_cell
            idf = plsc.load_gather(nnv, [one, lane_k, a_vec])
            sif = plsc.load_gather(nnv, [one + 1, lane_k, a_vec])
            valid = kvalid & (a_vec < _WD - 1) & (wid < _H - 1)
            sim_regs.append(jnp.where(valid, sif, jnp.float32(0.0)))
            idx_v[pl.ds(v * _LANES, _LANES)] = idf.astype(jnp.int32)
        cp0 = pltpu.async_copy(
            xw_hbm.at[idx_v.at[pl.ds(0, _GATHER_SPLIT)]],
            rows_v.at[pl.ds(0, _GATHER_SPLIT)], sem)
        cp1 = pltpu.async_copy(
            xw_hbm.at[idx_v.at[pl.ds(_GATHER_SPLIT, _GATHER_SPLIT)]],
            rows_v.at[pl.ds(_GATHER_SPLIT, _GATHER_SPLIT)], sem)
        cp0.wait()
        cp1.wait()
        for v in range(_ROWS_PER_W // _LANES):
            sv = sim_regs[v]
            for half, c in ((0, 2 * v), (_K, 2 * v + 1)):
                r0 = c * _K
                s = sv[half]
                acc_lo = s * rows_v[r0, pl.ds(0, _LANES)]
                acc_hi = s * rows_v[r0, pl.ds(_LANES, _LANES)]
                for k in range(1, _K):
                    r = r0 + k
                    s = sv[half + k]
                    acc_lo = acc_lo + s * rows_v[r, pl.ds(0, _LANES)]
                    acc_hi = acc_hi + s * rows_v[r, pl.ds(_LANES, _LANES)]
                out_v[c, pl.ds(0, _LANES)] = acc_lo
                out_v[c, pl.ds(_LANES, _LANES)] = acc_hi
        pltpu.sync_copy(out_v, out_hbm.at[wid])


_sc_gather = functools.partial(
    pl.kernel,
    out_type=jax.ShapeDtypeStruct((_H, _WD, _F), jnp.float32),
    mesh=plsc.VectorSubcoreMesh(core_axis_name="c", subcore_axis_name="s",
                                num_cores=_NC, num_subcores=_NS),
    scratch_types=[
        pltpu.VMEM((3, _K, _WD), jnp.float32),
        pltpu.VMEM((_ROWS_PER_W,), jnp.int32),
        pltpu.VMEM((_ROWS_PER_W, 128), jnp.float32),
        pltpu.VMEM((_CELLS_PER_W, _F), jnp.float32),
        pltpu.SemaphoreType.DMA,
    ],
    compiler_params=pltpu.CompilerParams(use_tc_tiling_on_sc=True,
                                         needs_layout_passes=False),
)(_sc_body)


def kernel(x, W, nearest_neighbors):
    xw = _project(x, W)
    return _sc_gather(xw, nearest_neighbors.transpose(0, 3, 2, 1))


# speedup vs baseline: 1.1519x; 1.0155x over previous
import functools

import jax
import jax.numpy as jnp
from jax import lax
from jax.experimental import pallas as pl
from jax.experimental.pallas import tpu as pltpu
from jax.experimental.pallas import tpu_sc as plsc

_H = 20
_WD = 20
_F = 32
_K = 8
_N = _H * _WD
_NC = 2
_NS = 16
_CELLS_PER_W = _WD
_ROWS_PER_W = _CELLS_PER_W * _K
_GATHER_SPLIT = 80
_LANES = 16
_SLAB = 24


def _mm_body(x_ref, w_ref, xw_ref):
    w = w_ref[...]
    for i in range(_H):
        xw_ref[pl.ds(i * _WD, _WD), pl.ds(0, _F)] = jnp.dot(
            x_ref[i], w, preferred_element_type=jnp.float32)


def _project(x, w):
    return pl.pallas_call(
        _mm_body,
        out_shape=jax.ShapeDtypeStruct((_N, 128), jnp.float32),
    )(x, w)


def _sc_body(xw_hbm, nn_hbm, out_hbm, nnv, idx_v, rows_v, out_v, sem):
    wid = lax.axis_index("s") * _NC + lax.axis_index("c")

    @pl.when(wid < _H)
    def _():
        pltpu.sync_copy(nn_hbm.at[wid], nnv)
        lane = lax.iota(jnp.int32, _LANES)
        lane_k = lane & (_K - 1)
        half_cell = lane >> 3
        kvalid = lane_k < _K - 1
        one = jnp.full((_LANES,), 1, jnp.int32)
        nv = _ROWS_PER_W // _LANES
        sim_regs = []

        def extract(v):
            a_vec = 2 * v + half_cell
            idf = plsc.load_gather(nnv, [one, lane_k, a_vec])
            sif = plsc.load_gather(nnv, [one + 1, lane_k, a_vec])
            valid = kvalid & (a_vec < _WD - 1) & (wid < _H - 1)
            sim_regs.append(jnp.where(valid, sif, jnp.float32(0.0)))
            idx_v[pl.ds(v * _LANES, _LANES)] = idf.astype(jnp.int32)

        def accumulate(v):
            sv = sim_regs[v]
            for half, c in ((0, 2 * v), (_K, 2 * v + 1)):
                r0 = c * _K
                s = sv[half]
                acc_lo = s * rows_v[r0, pl.ds(0, _LANES)]
                acc_hi = s * rows_v[r0, pl.ds(_LANES, _LANES)]
                for k in range(1, _K):
                    r = r0 + k
                    s = sv[half + k]
                    acc_lo = acc_lo + s * rows_v[r, pl.ds(0, _LANES)]
                    acc_hi = acc_hi + s * rows_v[r, pl.ds(_LANES, _LANES)]
                out_v[c, pl.ds(0, _LANES)] = acc_lo
                out_v[c, pl.ds(_LANES, _LANES)] = acc_hi

        for v in range(nv // 2):
            extract(v)
        cp0 = pltpu.async_copy(
            xw_hbm.at[idx_v.at[pl.ds(0, _GATHER_SPLIT)]],
            rows_v.at[pl.ds(0, _GATHER_SPLIT)], sem)
        for v in range(nv // 2, nv):
            extract(v)
        cp1 = pltpu.async_copy(
            xw_hbm.at[idx_v.at[pl.ds(_GATHER_SPLIT, _GATHER_SPLIT)]],
            rows_v.at[pl.ds(_GATHER_SPLIT, _GATHER_SPLIT)], sem)
        cp0.wait()
        for v in range(nv // 2):
            accumulate(v)
        cp1.wait()
        for v in range(nv // 2, nv):
            accumulate(v)
        pltpu.sync_copy(out_v, out_hbm.at[wid])


_sc_gather = functools.partial(
    pl.kernel,
    out_type=jax.ShapeDtypeStruct((_H, _WD, _F), jnp.float32),
    mesh=plsc.VectorSubcoreMesh(core_axis_name="c", subcore_axis_name="s",
                                num_cores=_NC, num_subcores=_NS),
    scratch_types=[
        pltpu.VMEM((3, _K, _WD), jnp.float32),
        pltpu.VMEM((_ROWS_PER_W,), jnp.int32),
        pltpu.VMEM((_ROWS_PER_W, 128), jnp.float32),
        pltpu.VMEM((_CELLS_PER_W, _F), jnp.float32),
        pltpu.SemaphoreType.DMA,
    ],
    compiler_params=pltpu.CompilerParams(use_tc_tiling_on_sc=True,
                                         needs_layout_passes=False),
)(_sc_body)


def kernel(x, W, nearest_neighbors):
    xw = _project(x, W)
    return _sc_gather(xw, nearest_neighbors.transpose(0, 3, 2, 1))
